# Initial kernel scaffold; baseline (speedup 1.0000x reference)
import jax
import jax.numpy as jnp
from jax.experimental import pallas as pl


def kernel(points):
    raise NotImplementedError("write your pallas kernel here")


# speedup vs baseline: 2.4686x; 2.4686x over previous
import functools

import jax
import jax.numpy as jnp
from jax import lax
from jax.experimental import pallas as pl
from jax.experimental.pallas import tpu as pltpu
import jax.experimental.pallas.tpu_sc as plsc

N = 120000
NPAD = 120064
TP = NPAD // 16
CH = 1792
NCH = NPAD // CH
GX, GY = 432, 496
BIG = GX * GY
CT = BIG // 16
CTP = 13440
MAX_VOXELS = 16000
MAX_POINTS = 32
NVOX_FLAT = MAX_VOXELS * MAX_POINTS
VS0, VS1, VS2 = 0.16, 0.16, 4.0
LO0, LO1, LO2 = 0.0, -39.68, -3.0

_mesh = plsc.VectorSubcoreMesh(core_axis_name="c", subcore_axis_name="s",
                               num_cores=1)

_out_type = (
    jax.ShapeDtypeStruct((NVOX_FLAT + 64,), jnp.float32),
    jax.ShapeDtypeStruct((NVOX_FLAT + 64,), jnp.float32),
    jax.ShapeDtypeStruct((NVOX_FLAT + 64,), jnp.float32),
    jax.ShapeDtypeStruct((NVOX_FLAT + 64,), jnp.float32),
    jax.ShapeDtypeStruct((MAX_VOXELS + 64,), jnp.int32),
    jax.ShapeDtypeStruct((MAX_VOXELS + 64,), jnp.int32),
    jax.ShapeDtypeStruct((MAX_VOXELS + 64,), jnp.int32),
)

_scratch = [
    pltpu.VMEM_SHARED((NPAD,), jnp.int32),
    pltpu.VMEM_SHARED((NPAD + 64,), jnp.int32),
    pltpu.VMEM_SHARED((256,), jnp.int32),
    pltpu.VMEM((CTP,), jnp.int32),
    pltpu.VMEM((CTP,), jnp.int32),
    pltpu.VMEM((CTP,), jnp.int32),
    pltpu.VMEM((CT,), jnp.int32),
    pltpu.VMEM((CH,), jnp.int32),
    pltpu.VMEM((CH,), jnp.float32),
    pltpu.VMEM((CH,), jnp.float32),
    pltpu.VMEM((CH,), jnp.float32),
    pltpu.VMEM((CH,), jnp.float32),
    pltpu.VMEM((128,), jnp.int32),
    pltpu.VMEM((128,), jnp.int32),
    pltpu.VMEM((128,), jnp.int32),
    pltpu.VMEM((128,), jnp.int32),
    pltpu.VMEM((128,), jnp.int32),
    pltpu.VMEM((128,), jnp.int32),
    pltpu.VMEM((128,), jnp.float32),
    pltpu.VMEM((128,), jnp.float32),
    pltpu.VMEM((128,), jnp.float32),
    pltpu.VMEM((128,), jnp.float32),
    pltpu.VMEM((16,), jnp.int32),
    pltpu.VMEM((16,), jnp.int32),
    pltpu.VMEM((16,), jnp.int32),
    pltpu.VMEM((256,), jnp.int32),
    pltpu.VMEM((TP,), jnp.int32),
    pltpu.VMEM((8192,), jnp.float32),
    pltpu.VMEM((2048,), jnp.int32),
    pltpu.SemaphoreType.DMA,
    pltpu.SemaphoreType.DMA,
]


@functools.partial(
    pl.kernel,
    out_type=_out_type,
    mesh=_mesh,
    compiler_params=pltpu.CompilerParams(needs_layout_passes=False),
    scratch_types=_scratch,
)
def _vox_kernel(xs, ys, zs, fs,
                voxx, voxy, voxz, voxf, counts, coorx, coory,
                sc_cells, sc_isf, sc_tot,
                cnt_grid, first_grid, rank_grid, tag_grid,
                cchunk, xch, ych, zch, fch,
                sbuf, idxbuf, ones128, cntval, cxval, cyval,
                vxval, vyval, vzval, vfval,
                tmp16, db_buf, last_buf, tot256, isfl, zf, zi, sem_z, sem_g):
    sid = lax.axis_index("s")
    base_pt = sid * TP
    cell_lo = sid * CT
    iota = lax.iota(jnp.int32, 16)
    tp_chunks = [(0, 1792), (1792, 1792), (3584, 1792), (5376, 1792),
                 (7168, 336)]

    def zfill(i, _):
        zf[pl.ds(16 * i, 16)] = jnp.zeros((16,), jnp.float32)
        return 0

    def zifill(i, _):
        zi[pl.ds(16 * i, 16)] = jnp.zeros((16,), jnp.int32)
        return 0

    lax.fori_loop(0, 512, zfill, 0)
    lax.fori_loop(0, 128, zifill, 0)

    for plane in (voxx, voxy, voxz, voxf):
        for k in range(4):
            sz = 8192 if k < 3 else NVOX_FLAT // 16 - 3 * 8192
            pltpu.sync_copy(
                zf.at[pl.ds(0, sz)],
                plane.at[pl.ds(sid * (NVOX_FLAT // 16) + k * 8192, sz)])
    for arr in (counts, coorx, coory):
        pltpu.sync_copy(
            zi.at[pl.ds(0, 1000)], arr.at[pl.ds(sid * 1000, 1000)])
    for k in range(4):
        sz = 2048 if k < 3 else TP - 3 * 2048
        pltpu.sync_copy(
            zi.at[pl.ds(0, sz)],
            sc_isf.at[pl.ds(base_pt + k * 2048, sz)])

    def ginit(i, _):
        cnt_grid[pl.ds(16 * i, 16)] = jnp.zeros((16,), jnp.int32)
        first_grid[pl.ds(16 * i, 16)] = jnp.full((16,), NPAD, jnp.int32)
        return 0

    lax.fori_loop(0, CTP // 16, ginit, 0)

    def floor_div(v, lo, vs):
        t = (v - lo) / vs
        i0 = t.astype(jnp.int32)
        return i0 - jnp.where(i0.astype(jnp.float32) > t, 1, 0)

    for off_c, sz in tp_chunks:
        pltpu.sync_copy(xs.at[pl.ds(base_pt + off_c, sz)], xch.at[pl.ds(0, sz)])
        pltpu.sync_copy(ys.at[pl.ds(base_pt + off_c, sz)], ych.at[pl.ds(0, sz)])
        pltpu.sync_copy(zs.at[pl.ds(base_pt + off_c, sz)], zch.at[pl.ds(0, sz)])

        def cbody(i, _):
            ix = floor_div(xch[pl.ds(16 * i, 16)], LO0, VS0)
            iy = floor_div(ych[pl.ds(16 * i, 16)], LO1, VS1)
            iz = floor_div(zch[pl.ds(16 * i, 16)], LO2, VS2)
            valid = ((ix >= 0) & (ix < GX) & (iy >= 0) & (iy < GY)
                     & (iz == 0))
            cchunk[pl.ds(16 * i, 16)] = jnp.where(valid, iy * GX + ix, BIG)
            return 0

        lax.fori_loop(0, sz // 16, cbody, 0)
        pltpu.sync_copy(cchunk.at[pl.ds(0, sz)],
                        sc_cells.at[pl.ds(base_pt + off_c, sz)])

    plsc.subcore_barrier()

    def slot_step(c, record_first, gidx):
        m = (c >= cell_lo) & (c < cell_lo + CT)
        lc = jnp.clip(c - cell_lo, 0, CT - 1)
        plsc.store_scatter(tag_grid, [lc], iota, mask=m)
        rb = plsc.load_gather(tag_grid, [lc], mask=m)
        anydup = jnp.any(m & (rb != iota))
        db_buf[...] = jnp.zeros((16,), jnp.int32)
        last_buf[...] = jnp.where(m, 1, 0)

        @pl.when(anydup)
        def _():
            mi = jnp.where(m, 1, 0)
            db = jnp.zeros((16,), jnp.int32)
            notlast = jnp.zeros((16,), jnp.int32)
            for s in range(1, 16):
                ib = jnp.maximum(iota - s, 0)
                eqb = ((lc == lc[ib]) & (mi[ib] == 1) & (iota >= s) & m)
                db = db + jnp.where(eqb, 1, 0)
                jf = jnp.minimum(iota + s, 15)
                eqf = ((lc == lc[jf]) & (mi[jf] == 1) & (iota + s <= 15) & m)
                notlast = notlast | jnp.where(eqf, 1, 0)
            db_buf[...] = db
            last_buf[...] = jnp.where(m & (notlast == 0), 1, 0)

        db = db_buf[...]
        lastm = last_buf[...] != 0
        cnt = plsc.load_gather(cnt_grid, [lc], mask=m)
        slot = cnt + db
        plsc.store_scatter(cnt_grid, [lc], slot + 1, mask=lastm)
        isf = m & (slot == 0)
        if record_first:
            plsc.store_scatter(first_grid, [lc], gidx, mask=isf)
        return m, lc, slot, isf

    def fill_idxbuf(dummy):
        for k in range(8):
            idxbuf[pl.ds(16 * k, 16)] = jnp.full((16,), dummy, jnp.int32)

    def fill_ones(i, _):
        ones128[pl.ds(16 * i, 16)] = jnp.ones((16,), jnp.int32)
        return 0

    lax.fori_loop(0, 8, fill_ones, 0)
    fill_idxbuf(NPAD)

    def p1_chunk(ch_i, off):
        pltpu.sync_copy(sc_cells.at[pl.ds(ch_i * CH, CH)], cchunk)

        def p1_step(u, off2):
            gidx = ch_i * CH + 16 * u + iota
            c = cchunk[pl.ds(16 * u, 16)]
            m, lc, slot, isf = slot_step(c, True, gidx)
            pos = plsc.cumsum(jnp.where(isf, 1, 0)) - 1
            plsc.store_scatter(idxbuf, [off2 + pos], gidx, mask=isf)
            noff = off2 + jnp.sum(jnp.where(isf, 1, 0))

            @pl.when(noff >= 112)
            def _():
                pltpu.sync_copy(ones128, sc_isf.at[idxbuf])
                fill_idxbuf(NPAD)

            return jnp.where(noff >= 112, 0, noff)

        return lax.fori_loop(0, CH // 16, p1_step, off)

    lax.fori_loop(0, NCH, p1_chunk, jnp.int32(0))
    pltpu.sync_copy(ones128, sc_isf.at[idxbuf])
    plsc.subcore_barrier()

    pltpu.sync_copy(sc_isf.at[pl.ds(base_pt, TP)], isfl)

    def p2_body(i, cr):
        v = isfl[pl.ds(16 * i, 16)]
        isfl[pl.ds(16 * i, 16)] = plsc.cumsum(v) + cr
        return cr + jnp.sum(v)

    my_tot = lax.fori_loop(0, TP // 16, p2_body, jnp.int32(0))
    tmp16[...] = jnp.full((16,), my_tot, jnp.int32)
    pltpu.sync_copy(tmp16, sc_tot.at[pl.ds(sid * 16, 16)])
    plsc.subcore_barrier()
    pltpu.sync_copy(sc_tot, tot256)
    tot16 = plsc.load_gather(tot256, [iota * 16], mask=None)
    excl = jnp.sum(jnp.where(iota < sid, tot16, 0))

    def p2_add(i, _):
        isfl[pl.ds(16 * i, 16)] = isfl[pl.ds(16 * i, 16)] + excl
        return 0

    lax.fori_loop(0, TP // 16, p2_add, 0)
    pltpu.sync_copy(isfl, sc_isf.at[pl.ds(base_pt, TP)])
    plsc.subcore_barrier()

    fill_idxbuf(MAX_VOXELS)

    def p3_chunk(cb, off):
        pltpu.async_copy(
            sc_isf.at[first_grid.at[pl.ds(cb * 128, 128)]], sbuf,
            sem_g).wait()

        def p3_step(u, off2):
            j = cb * 128 + 16 * u
            cnt = cnt_grid[pl.ds(j, 16)]
            m = cnt > 0
            rank = sbuf[pl.ds(16 * u, 16)] - 1
            rank_grid[pl.ds(j, 16)] = jnp.where(m, rank, NVOX_FLAT)
            keep = m & (rank < MAX_VOXELS)
            gcell = cell_lo + j + iota
            cx = lax.rem(gcell, GX)
            cy = lax.div(gcell, GX)
            pos = plsc.cumsum(jnp.where(keep, 1, 0)) - 1
            dst = off2 + pos
            plsc.store_scatter(idxbuf, [dst], rank, mask=keep)
            plsc.store_scatter(cntval, [dst], jnp.minimum(cnt, MAX_POINTS),
                               mask=keep)
            plsc.store_scatter(cxval, [dst], cx, mask=keep)
            plsc.store_scatter(cyval, [dst], cy, mask=keep)
            noff = off2 + jnp.sum(jnp.where(keep, 1, 0))

            @pl.when(noff >= 112)
            def _():
                pltpu.sync_copy(cntval, counts.at[idxbuf])
                pltpu.sync_copy(cxval, coorx.at[idxbuf])
                pltpu.sync_copy(cyval, coory.at[idxbuf])
                fill_idxbuf(MAX_VOXELS)

            return jnp.where(noff >= 112, 0, noff)

        return lax.fori_loop(0, 8, p3_step, off)

    lax.fori_loop(0, CTP // 128, p3_chunk, jnp.int32(0))
    pltpu.sync_copy(cntval, counts.at[idxbuf])
    pltpu.sync_copy(cxval, coorx.at[idxbuf])
    pltpu.sync_copy(cyval, coory.at[idxbuf])

    def gzero(i, _):
        cnt_grid[pl.ds(16 * i, 16)] = jnp.zeros((16,), jnp.int32)
        return 0

    lax.fori_loop(0, CTP // 16, gzero, 0)
    fill_idxbuf(NVOX_FLAT)

    def p4_chunk(ch_i, off):
        pltpu.sync_copy(sc_cells.at[pl.ds(ch_i * CH, CH)], cchunk)
        pltpu.sync_copy(xs.at[pl.ds(ch_i * CH, CH)], xch)
        pltpu.sync_copy(ys.at[pl.ds(ch_i * CH, CH)], ych)
        pltpu.sync_copy(zs.at[pl.ds(ch_i * CH, CH)], zch)
        pltpu.sync_copy(fs.at[pl.ds(ch_i * CH, CH)], fch)

        def p4_step(u, off2):
            c = cchunk[pl.ds(16 * u, 16)]
            m, lc, slot, _ = slot_step(c, False, None)
            rank = plsc.load_gather(rank_grid, [lc], mask=m)
            keep = m & (slot < MAX_POINTS) & (rank < MAX_VOXELS)
            tidx = rank * MAX_POINTS + slot
            pos = plsc.cumsum(jnp.where(keep, 1, 0)) - 1
            dst = off2 + pos
            plsc.store_scatter(idxbuf, [dst], tidx, mask=keep)
            plsc.store_scatter(vxval, [dst], xch[pl.ds(16 * u, 16)], mask=keep)
            plsc.store_scatter(vyval, [dst], ych[pl.ds(16 * u, 16)], mask=keep)
            plsc.store_scatter(vzval, [dst], zch[pl.ds(16 * u, 16)], mask=keep)
            plsc.store_scatter(vfval, [dst], fch[pl.ds(16 * u, 16)], mask=keep)
            noff = off2 + jnp.sum(jnp.where(keep, 1, 0))

            @pl.when(noff >= 112)
            def _():
                pltpu.sync_copy(vxval, voxx.at[idxbuf])
                pltpu.sync_copy(vyval, voxy.at[idxbuf])
                pltpu.sync_copy(vzval, voxz.at[idxbuf])
                pltpu.sync_copy(vfval, voxf.at[idxbuf])
                fill_idxbuf(NVOX_FLAT)

            return jnp.where(noff >= 112, 0, noff)

        return lax.fori_loop(0, CH // 16, p4_step, off)

    lax.fori_loop(0, NCH, p4_chunk, jnp.int32(0))
    pltpu.sync_copy(vxval, voxx.at[idxbuf])
    pltpu.sync_copy(vyval, voxy.at[idxbuf])
    pltpu.sync_copy(vzval, voxz.at[idxbuf])
    pltpu.sync_copy(vfval, voxf.at[idxbuf])


def kernel(points):
    pts = points.astype(jnp.float32)
    pad = jnp.full((NPAD - N,), -1e6, jnp.float32)
    xs = jnp.concatenate([pts[:, 0], pad])
    ys = jnp.concatenate([pts[:, 1], pad])
    zs = jnp.concatenate([pts[:, 2], pad])
    fs = jnp.concatenate([pts[:, 3], pad])
    px, py, pz, pf, counts, coorx, coory = _vox_kernel(xs, ys, zs, fs)
    voxels = jnp.stack(
        [px[:NVOX_FLAT].reshape(MAX_VOXELS, MAX_POINTS),
         py[:NVOX_FLAT].reshape(MAX_VOXELS, MAX_POINTS),
         pz[:NVOX_FLAT].reshape(MAX_VOXELS, MAX_POINTS),
         pf[:NVOX_FLAT].reshape(MAX_VOXELS, MAX_POINTS)],
        axis=-1)
    coors = jnp.stack(
        [coorx[:MAX_VOXELS], coory[:MAX_VOXELS],
         jnp.zeros((MAX_VOXELS,), jnp.int32)],
        axis=1)
    return voxels, coors, counts[:MAX_VOXELS]
